# baseline (speedup 1.0000x reference)
import jax
import jax.numpy as jnp
from jax.experimental import pallas as pl
from jax.experimental.pallas import tpu as pltpu

B, S, D = 4, 4096, 2048
R = B * S
BLK = 512


def _body(mask_ref, mte_ref, emb_ref, out_ref):
    out_ref[...] = jnp.where(mask_ref[...] != 0, mte_ref[...], emb_ref[...])


def kernel(embeddings, token_mask, mask_token_embed):
    emb = embeddings.reshape(R, D)
    mask = token_mask.reshape(R, 1).astype(jnp.int32)
    mte = mask_token_embed.reshape(1, D)

    out = pl.pallas_call(
        _body,
        grid=(R // BLK,),
        in_specs=[
            pl.BlockSpec((BLK, 1), lambda i: (i, 0)),
            pl.BlockSpec((1, D), lambda i: (0, 0)),
            pl.BlockSpec((BLK, D), lambda i: (i, 0)),
        ],
        out_specs=pl.BlockSpec((BLK, D), lambda i: (i, 0)),
        out_shape=jax.ShapeDtypeStruct((R, D), jnp.float32),
        compiler_params=pltpu.CompilerParams(
            dimension_semantics=("arbitrary",),
        ),
    )(mask, mte, emb)
    return out.reshape(B, S, D)
